# Initial kernel scaffold; baseline (speedup 1.0000x reference)
import jax
import jax.numpy as jnp
from jax.experimental import pallas as pl


def kernel(item_feats, user_feats, edge_index):
    raise NotImplementedError("write your pallas kernel here")


# speedup vs baseline: 1.1415x; 1.1415x over previous
import functools

import jax
import jax.numpy as jnp
from jax import lax
from jax.experimental import pallas as pl
from jax.experimental.pallas import tpu as pltpu
from jax.experimental.pallas import tpu_sc as plsc

N_ITEM = 10000
N_USER = 10000
E = 160000
D = 256

NC = 2
NS = 16
NW = NC * NS
E_PAD = 163840
E_PER_W = E_PAD // NW
C = 128
N_CHUNKS = E_PER_W // C
G = C // 16


def _sc_body(item_hbm, user_hbm, src_hbm, dst_hbm, out_hbm,
             src_v, dst_v, u_buf, v_buf, out_v, sem):
  wid = lax.axis_index("s") * NC + lax.axis_index("c")
  w_base = wid * E_PER_W

  def chunk_body(c, carry):
    base = w_base + c * C
    pltpu.sync_copy(src_hbm.at[pl.ds(base, C)], src_v)
    pltpu.sync_copy(dst_hbm.at[pl.ds(base, C)], dst_v)
    pltpu.async_copy(item_hbm.at[src_v], u_buf, sem).wait()
    pltpu.async_copy(user_hbm.at[dst_v], v_buf, sem).wait()
    zeros16 = jnp.zeros((16,), jnp.float32)
    for g in range(G):
      out_v[pl.ds(g * 16, 16)] = zeros16

    def e_body(e, carry2):
      acc = u_buf[e, pl.ds(0, 16)] * v_buf[e, pl.ds(0, 16)]
      for i in range(1, D // 16):
        acc += u_buf[e, pl.ds(i * 16, 16)] * v_buf[e, pl.ds(i * 16, 16)]
      plsc.addupdate_scatter(out_v, [jnp.full((16,), e, jnp.int32)], acc)
      return carry2

    lax.fori_loop(0, C, e_body, jnp.int32(0), unroll=2)
    pltpu.sync_copy(out_v, out_hbm.at[pl.ds(base, C)])
    return carry

  lax.fori_loop(0, N_CHUNKS, chunk_body, jnp.int32(0))


_sc_call = functools.partial(
    pl.kernel,
    out_type=jax.ShapeDtypeStruct((E_PAD,), jnp.float32),
    mesh=plsc.VectorSubcoreMesh(core_axis_name="c", subcore_axis_name="s"),
    compiler_params=pltpu.CompilerParams(
        use_tc_tiling_on_sc=False, needs_layout_passes=False),
    scratch_types=[
        pltpu.VMEM((C,), jnp.int32),
        pltpu.VMEM((C,), jnp.int32),
        pltpu.VMEM((C, D), jnp.float32),
        pltpu.VMEM((C, D), jnp.float32),
        pltpu.VMEM((C,), jnp.float32),
        pltpu.SemaphoreType.DMA,
    ],
)(_sc_body)


@jax.jit
def kernel(item_feats, user_feats, edge_index):
  src = edge_index[0].astype(jnp.int32)
  dst = edge_index[1].astype(jnp.int32)
  pad = E_PAD - E
  src = jnp.concatenate([src, jnp.zeros((pad,), jnp.int32)])
  dst = jnp.concatenate([dst, jnp.zeros((pad,), jnp.int32)])
  out = _sc_call(item_feats, user_feats, src, dst)
  return out[:E]


# speedup vs baseline: 3.5439x; 3.1047x over previous
import functools

import jax
import jax.numpy as jnp
from jax import lax
from jax.experimental import pallas as pl
from jax.experimental.pallas import tpu as pltpu
from jax.experimental.pallas import tpu_sc as plsc

N_ITEM = 10000
N_USER = 10000
E = 160000
D = 256

NC = 2
NS = 16
NW = NC * NS
E_PER_W = E // NW
C = 40
N_CHUNKS = E_PER_W // C
NBUF = 3
OUT_PAD = 5008


def _sc_body(item_hbm, user_hbm, src_hbm, dst_hbm, out_hbm,
             src_all, dst_all, out_all, u_bufs, v_bufs, sems):
  wid = lax.axis_index("s") * NC + lax.axis_index("c")
  w_base = wid * E_PER_W

  pltpu.sync_copy(src_hbm.at[pl.ds(w_base, E_PER_W)], src_all)
  pltpu.sync_copy(dst_hbm.at[pl.ds(w_base, E_PER_W)], dst_all)

  zeros16 = jnp.zeros((16,), jnp.float32)

  def zero_body(i, carry):
    out_all[pl.ds(i * 16, 16)] = zeros16
    return carry

  lax.fori_loop(0, OUT_PAD // 16, zero_body, jnp.int32(0), unroll=4)

  def start(c, b):
    pltpu.async_copy(item_hbm.at[src_all.at[pl.ds(c * C, C)]],
                     u_bufs[b], sems[b])
    pltpu.async_copy(user_hbm.at[dst_all.at[pl.ds(c * C, C)]],
                     v_bufs[b], sems[b])

  def wait(c, b):
    pltpu.make_async_copy(item_hbm.at[src_all.at[pl.ds(c * C, C)]],
                          u_bufs[b], sems[b]).wait()
    pltpu.make_async_copy(user_hbm.at[dst_all.at[pl.ds(c * C, C)]],
                          v_bufs[b], sems[b]).wait()

  def compute(c, b):
    u_buf = u_bufs[b]
    v_buf = v_bufs[b]
    out_base = c * C

    def e_body(e, carry2):
      acc = u_buf[e, pl.ds(0, 16)] * v_buf[e, pl.ds(0, 16)]
      for i in range(1, D // 16):
        acc += u_buf[e, pl.ds(i * 16, 16)] * v_buf[e, pl.ds(i * 16, 16)]
      plsc.addupdate_scatter(
          out_all, [jnp.full((16,), out_base + e, jnp.int32)], acc)
      return carry2

    lax.fori_loop(0, C, e_body, jnp.int32(0), unroll=2)

  for b in range(NBUF - 1):
    start(jnp.int32(b), b)

  def outer(i, carry):
    for p in range(NBUF):
      c = i * NBUF + p

      @pl.when(c + NBUF - 1 < N_CHUNKS)
      def _():
        start(c + NBUF - 1, (p + NBUF - 1) % NBUF)

      @pl.when(c < N_CHUNKS)
      def _():
        wait(c, p)
        compute(c, p)
    return carry

  n_outer = (N_CHUNKS + NBUF - 1) // NBUF
  lax.fori_loop(0, n_outer, outer, jnp.int32(0))

  pltpu.sync_copy(out_all.at[pl.ds(0, E_PER_W)],
                  out_hbm.at[pl.ds(w_base, E_PER_W)])


_sc_call = functools.partial(
    pl.kernel,
    out_type=jax.ShapeDtypeStruct((E,), jnp.float32),
    mesh=plsc.VectorSubcoreMesh(core_axis_name="c", subcore_axis_name="s"),
    compiler_params=pltpu.CompilerParams(
        use_tc_tiling_on_sc=False, needs_layout_passes=False),
    scratch_types=[
        pltpu.VMEM((E_PER_W,), jnp.int32),
        pltpu.VMEM((E_PER_W,), jnp.int32),
        pltpu.VMEM((OUT_PAD,), jnp.float32),
        [pltpu.VMEM((C, D), jnp.float32) for _ in range(NBUF)],
        [pltpu.VMEM((C, D), jnp.float32) for _ in range(NBUF)],
        [pltpu.SemaphoreType.DMA for _ in range(NBUF)],
    ],
)(_sc_body)


@jax.jit
def kernel(item_feats, user_feats, edge_index):
  src = edge_index[0].astype(jnp.int32)
  dst = edge_index[1].astype(jnp.int32)
  return _sc_call(item_feats, user_feats, src, dst)
